# Initial kernel scaffold; baseline (speedup 1.0000x reference)
import jax
import jax.numpy as jnp
from jax.experimental import pallas as pl


def kernel(node_x, node_y, h_edges, v_edges):
    raise NotImplementedError("write your pallas kernel here")


# speedup vs baseline: 212.6976x; 212.6976x over previous
import functools

import jax
import jax.numpy as jnp
from jax import lax
from jax.experimental import pallas as pl
from jax.experimental.pallas import tpu as pltpu
from jax.experimental.pallas import tpu_sc as plsc

_N = 100000
_E = 1600000
_NW = 32
_EPW = _E // _NW
_CHUNK = 2000


def _make_kernel():
    mesh = plsc.VectorSubcoreMesh(core_axis_name="c", subcore_axis_name="s")

    @functools.partial(
        pl.kernel,
        out_type=jax.ShapeDtypeStruct((_NW, 16), jnp.float32),
        mesh=mesh,
        compiler_params=pltpu.CompilerParams(needs_layout_passes=False),
        scratch_types=[
            pltpu.VMEM((_N,), jnp.float32),
            pltpu.VMEM((_CHUNK,), jnp.int32),
            pltpu.VMEM((_CHUNK,), jnp.int32),
            pltpu.VMEM((16,), jnp.float32),
        ],
    )
    def k(node_x, node_y, h0, h1, v0, v1, out, table_v, i0_v, i1_v, acc_v):
        wid = lax.axis_index("s") * 2 + lax.axis_index("c")
        base = pl.multiple_of(wid * _EPW, 8)

        def phase(nodes_hbm, e0_hbm, e1_hbm, acc):
            pltpu.sync_copy(nodes_hbm, table_v)

            def chunk_body(ci, acc):
                off = pl.multiple_of(base + ci * _CHUNK, 8)
                pltpu.sync_copy(e0_hbm.at[pl.ds(off, _CHUNK)], i0_v)
                pltpu.sync_copy(e1_hbm.at[pl.ds(off, _CHUNK)], i1_v)

                def inner(j, acc):
                    s = pl.multiple_of(j * 16, 16)
                    a = plsc.load_gather(table_v, [i0_v[pl.ds(s, 16)]])
                    b = plsc.load_gather(table_v, [i1_v[pl.ds(s, 16)]])
                    return acc + jnp.abs(a - b)

                return lax.fori_loop(0, _CHUNK // 16, inner, acc)

            return lax.fori_loop(0, _EPW // _CHUNK, chunk_body, acc)

        acc = jnp.zeros((16,), jnp.float32)
        acc = phase(node_x, h0, h1, acc)
        acc = phase(node_y, v0, v1, acc)
        acc_v[...] = acc
        pltpu.sync_copy(acc_v, out.at[wid])

    return k


_sc_kernel = _make_kernel()


def kernel(node_x, node_y, h_edges, v_edges):
    partials = _sc_kernel(
        node_x, node_y,
        h_edges[0], h_edges[1], v_edges[0], v_edges[1],
    )
    return jnp.sum(partials)


# speedup vs baseline: 238.9656x; 1.1235x over previous
import functools

import jax
import jax.numpy as jnp
from jax import lax
from jax.experimental import pallas as pl
from jax.experimental.pallas import tpu as pltpu
from jax.experimental.pallas import tpu_sc as plsc

_N = 100000
_E = 1600000
_NS = 16
_EPW = _E // _NS
_CHUNK = 2000
_GROUPS = _CHUNK // 16
_U = 5


def _make_kernel():
    mesh = plsc.VectorSubcoreMesh(core_axis_name="c", subcore_axis_name="s")

    @functools.partial(
        pl.kernel,
        out_type=jax.ShapeDtypeStruct((32, 16), jnp.float32),
        mesh=mesh,
        compiler_params=pltpu.CompilerParams(needs_layout_passes=False),
        scratch_types=[
            pltpu.VMEM((_N,), jnp.float32),
            pltpu.VMEM((_CHUNK,), jnp.int32),
            pltpu.VMEM((_CHUNK,), jnp.int32),
            pltpu.VMEM((16,), jnp.float32),
        ],
    )
    def k(node_x, node_y, h0, h1, v0, v1, out, table_v, i0_v, i1_v, acc_v):
        cid = lax.axis_index("c")
        sid = lax.axis_index("s")
        base = pl.multiple_of(sid * _EPW, 8)

        def phase(nodes_hbm, e0_hbm, e1_hbm):
            pltpu.sync_copy(nodes_hbm, table_v)

            def chunk_body(ci, accs):
                off = pl.multiple_of(base + ci * _CHUNK, 8)
                pltpu.sync_copy(e0_hbm.at[pl.ds(off, _CHUNK)], i0_v)
                pltpu.sync_copy(e1_hbm.at[pl.ds(off, _CHUNK)], i1_v)

                def inner(j, accs):
                    a0, a1 = accs
                    for u in range(_U):
                        s = pl.multiple_of((j * _U + u) * 16, 16)
                        a = plsc.load_gather(table_v, [i0_v[pl.ds(s, 16)]])
                        b = plsc.load_gather(table_v, [i1_v[pl.ds(s, 16)]])
                        t = jnp.abs(a - b)
                        if u % 2 == 0:
                            a0 = a0 + t
                        else:
                            a1 = a1 + t
                    return (a0, a1)

                return lax.fori_loop(0, _GROUPS // _U, inner, accs)

            z = jnp.zeros((16,), jnp.float32)
            accs = lax.fori_loop(0, _EPW // _CHUNK, chunk_body, (z, z))
            acc_v[...] = accs[0] + accs[1]

        @pl.when(cid == 0)
        def _():
            phase(node_x, h0, h1)

        @pl.when(cid == 1)
        def _():
            phase(node_y, v0, v1)

        pltpu.sync_copy(acc_v, out.at[sid * 2 + cid])

    return k


_sc_kernel = _make_kernel()


def kernel(node_x, node_y, h_edges, v_edges):
    partials = _sc_kernel(
        node_x, node_y,
        h_edges[0], h_edges[1], v_edges[0], v_edges[1],
    )
    return jnp.sum(partials)


# speedup vs baseline: 614.4188x; 2.5712x over previous
import functools

import jax
import jax.numpy as jnp
from jax import lax
from jax.experimental import pallas as pl
from jax.experimental.pallas import tpu as pltpu
from jax.experimental.pallas import tpu_sc as plsc

_N = 100000
_E = 1600000
_NS = 16
_CHUNK = 3200
_NCHUNK = _E // _CHUNK
_GROUPS = _CHUNK // 16
_U = 8


def _make_kernel():
    mesh = plsc.VectorSubcoreMesh(core_axis_name="c", subcore_axis_name="s")

    @functools.partial(
        pl.kernel,
        out_type=jax.ShapeDtypeStruct((32, 16), jnp.float32),
        mesh=mesh,
        compiler_params=pltpu.CompilerParams(needs_layout_passes=False),
        scratch_types=[
            pltpu.VMEM((_N,), jnp.float32),
            pltpu.VMEM((2, _CHUNK), jnp.int32),
            pltpu.VMEM((16,), jnp.float32),
        ],
    )
    def k(node_x, node_y, h_edges, v_edges, out, table_v, idx_v, acc_v):
        cid = lax.axis_index("c")
        sid = lax.axis_index("s")
        c_lo = (_NCHUNK * sid) // _NS
        c_hi = (_NCHUNK * (sid + 1)) // _NS

        def phase(nodes_hbm, edges_hbm):
            pltpu.sync_copy(nodes_hbm, table_v)

            def chunk_body(ci, accs):
                off = pl.multiple_of(ci * _CHUNK, 128)
                pltpu.sync_copy(edges_hbm.at[:, pl.ds(off, _CHUNK)], idx_v)

                @plsc.parallel_loop(0, _GROUPS, unroll=_U, carry=accs)
                def inner(j, accs):
                    a0, a1 = accs
                    s = pl.multiple_of(j * 16, 16)
                    a = plsc.load_gather(table_v, [idx_v[0, pl.ds(s, 16)]])
                    b = plsc.load_gather(table_v, [idx_v[1, pl.ds(s, 16)]])
                    t = jnp.abs(a - b)
                    return (a1, a0 + t)

                return inner

            z = jnp.zeros((16,), jnp.float32)
            accs = lax.fori_loop(c_lo, c_hi, chunk_body, (z, z))
            acc_v[...] = accs[0] + accs[1]

        @pl.when(cid == 0)
        def _():
            phase(node_x, h_edges)

        @pl.when(cid == 1)
        def _():
            phase(node_y, v_edges)

        pltpu.sync_copy(acc_v, out.at[sid * 2 + cid])

    return k


_sc_kernel = _make_kernel()


def kernel(node_x, node_y, h_edges, v_edges):
    partials = _sc_kernel(node_x, node_y, h_edges, v_edges)
    return jnp.sum(partials)


# speedup vs baseline: 849.0037x; 1.3818x over previous
import functools

import jax
import jax.numpy as jnp
from jax import lax
from jax.experimental import pallas as pl
from jax.experimental.pallas import tpu as pltpu
from jax.experimental.pallas import tpu_sc as plsc

_N = 100000
_E = 1600000
_NS = 16
_CHUNK = 3200
_NCHUNK = _E // _CHUNK
_SCHED = 32
_GROUPS = _CHUNK // 16
_U = 8


def _make_kernel():
    mesh = plsc.VectorSubcoreMesh(core_axis_name="c", subcore_axis_name="s")

    @functools.partial(
        pl.kernel,
        out_type=jax.ShapeDtypeStruct((32, 16), jnp.float32),
        mesh=mesh,
        compiler_params=pltpu.CompilerParams(needs_layout_passes=False),
        scratch_types=[
            pltpu.VMEM((_N,), jnp.float32),
            pltpu.VMEM((2, 2, _CHUNK), jnp.int32),
            pltpu.VMEM((16,), jnp.float32),
            pltpu.SemaphoreType.DMA,
            pltpu.SemaphoreType.DMA,
            pltpu.SemaphoreType.DMA,
        ],
    )
    def k(node_x, node_y, h_edges, v_edges, out,
          table_v, idx_v, acc_v, tsem, sem0, sem1):
        cid = lax.axis_index("c")
        sid = lax.axis_index("s")
        c_lo = (_NCHUNK * sid) // _NS
        c_hi = (_NCHUNK * (sid + 1)) // _NS
        sems = (sem0, sem1)

        def phase(nodes_hbm, edges_hbm):
            def src(g):
                c = jnp.minimum(c_lo + g, c_hi - 1)
                off = pl.multiple_of(c * _CHUNK, 128)
                return edges_hbm.at[:, pl.ds(off, _CHUNK)]

            def start(b, g):
                pltpu.async_copy(src(g), idx_v.at[b], sems[b])

            def wait(b):
                pltpu.make_async_copy(src(0), idx_v.at[b], sems[b]).wait()

            def compute(b, g, accs):
                @plsc.parallel_loop(0, _GROUPS, unroll=_U,
                                    carry=(jnp.zeros((16,), jnp.float32),
                                           jnp.zeros((16,), jnp.float32)))
                def csum(j, cc):
                    c0, c1 = cc
                    s = pl.multiple_of(j * 16, 16)
                    a = plsc.load_gather(table_v, [idx_v[b, 0, pl.ds(s, 16)]])
                    bb = plsc.load_gather(table_v, [idx_v[b, 1, pl.ds(s, 16)]])
                    return (c1, c0 + jnp.abs(a - bb))

                live = c_lo + g < c_hi
                a0, a1 = accs
                s0, s1 = csum
                return (a0 + jnp.where(live, s0, 0.0),
                        a1 + jnp.where(live, s1, 0.0))

            tcp = pltpu.async_copy(nodes_hbm, table_v, tsem)
            start(0, 0)
            tcp.wait()

            def pair_body(g2, accs):
                ga = 2 * g2
                start(1, ga + 1)
                wait(0)
                accs = compute(0, ga, accs)

                @pl.when(ga + 2 < _SCHED)
                def _():
                    start(0, ga + 2)

                wait(1)
                return compute(1, ga + 1, accs)

            z = jnp.zeros((16,), jnp.float32)
            accs = lax.fori_loop(0, _SCHED // 2, pair_body, (z, z))
            acc_v[...] = accs[0] + accs[1]

        @pl.when(cid == 0)
        def _():
            phase(node_x, h_edges)

        @pl.when(cid == 1)
        def _():
            phase(node_y, v_edges)

        pltpu.sync_copy(acc_v, out.at[sid * 2 + cid])

    return k


_sc_kernel = _make_kernel()


def kernel(node_x, node_y, h_edges, v_edges):
    partials = _sc_kernel(node_x, node_y, h_edges, v_edges)
    return jnp.sum(partials)


# speedup vs baseline: 928.4718x; 1.0936x over previous
import functools

import jax
import jax.numpy as jnp
from jax import lax
from jax.experimental import pallas as pl
from jax.experimental.pallas import tpu as pltpu
from jax.experimental.pallas import tpu_sc as plsc

_N = 100000
_E = 1600000
_NS = 16
_CHUNK = 6400
_NCHUNK = _E // _CHUNK
_SCHED = 16
_GROUPS = _CHUNK // 16
_U = 16


def _make_kernel():
    mesh = plsc.VectorSubcoreMesh(core_axis_name="c", subcore_axis_name="s")

    @functools.partial(
        pl.kernel,
        out_type=jax.ShapeDtypeStruct((32, 16), jnp.float32),
        mesh=mesh,
        compiler_params=pltpu.CompilerParams(needs_layout_passes=False),
        scratch_types=[
            pltpu.VMEM((_N,), jnp.float32),
            pltpu.VMEM((2, 2, _CHUNK), jnp.int32),
            pltpu.VMEM((16,), jnp.float32),
            pltpu.SemaphoreType.DMA,
            pltpu.SemaphoreType.DMA,
            pltpu.SemaphoreType.DMA,
        ],
    )
    def k(node_x, node_y, h_edges, v_edges, out,
          table_v, idx_v, acc_v, tsem, sem0, sem1):
        cid = lax.axis_index("c")
        sid = lax.axis_index("s")
        c_lo = (_NCHUNK * sid) // _NS
        c_hi = (_NCHUNK * (sid + 1)) // _NS
        sems = (sem0, sem1)

        def phase(nodes_hbm, edges_hbm):
            def src(g):
                c = jnp.minimum(c_lo + g, c_hi - 1)
                off = pl.multiple_of(c * _CHUNK, 128)
                return edges_hbm.at[:, pl.ds(off, _CHUNK)]

            def start(b, g):
                pltpu.async_copy(src(g), idx_v.at[b], sems[b])

            def wait(b):
                pltpu.make_async_copy(src(0), idx_v.at[b], sems[b]).wait()

            def compute(b, g, accs):
                @plsc.parallel_loop(0, _GROUPS, unroll=_U,
                                    carry=(jnp.zeros((16,), jnp.float32),
                                           jnp.zeros((16,), jnp.float32)))
                def csum(j, cc):
                    c0, c1 = cc
                    s = pl.multiple_of(j * 16, 16)
                    a = plsc.load_gather(table_v, [idx_v[b, 0, pl.ds(s, 16)]])
                    bb = plsc.load_gather(table_v, [idx_v[b, 1, pl.ds(s, 16)]])
                    return (c1, c0 + jnp.abs(a - bb))

                live = c_lo + g < c_hi
                a0, a1 = accs
                s0, s1 = csum
                return (a0 + jnp.where(live, s0, 0.0),
                        a1 + jnp.where(live, s1, 0.0))

            tcp = pltpu.async_copy(nodes_hbm, table_v, tsem)
            start(0, 0)
            tcp.wait()

            def pair_body(g2, accs):
                ga = 2 * g2
                start(1, ga + 1)
                wait(0)
                accs = compute(0, ga, accs)

                @pl.when(ga + 2 < _SCHED)
                def _():
                    start(0, ga + 2)

                wait(1)
                return compute(1, ga + 1, accs)

            z = jnp.zeros((16,), jnp.float32)
            accs = lax.fori_loop(0, _SCHED // 2, pair_body, (z, z))
            acc_v[...] = accs[0] + accs[1]

        @pl.when(cid == 0)
        def _():
            phase(node_x, h_edges)

        @pl.when(cid == 1)
        def _():
            phase(node_y, v_edges)

        pltpu.sync_copy(acc_v, out.at[sid * 2 + cid])

    return k


_sc_kernel = _make_kernel()


def kernel(node_x, node_y, h_edges, v_edges):
    partials = _sc_kernel(node_x, node_y, h_edges, v_edges)
    return jnp.sum(partials)
